# baseline (speedup 1.0000x reference)
import functools

import jax
import jax.numpy as jnp
from jax import lax
from jax.experimental import pallas as pl
from jax.experimental.pallas import tpu as pltpu
from jax.experimental.pallas import tpu_sc as plsc

_NW = 32
_LANES = 16
_CH = 64
_N_GRID = 512
_TILE = _CH * _N_GRID


def _sc_body(g0inv_hbm, zeros_hbm, params_hbm, out_hbm,
             g0inv_v, params_v, buf0, buf1, idx0, idx1, sem0, sem1):
    cid = lax.axis_index("c")
    sid = lax.axis_index("s")
    wid = sid * 2 + cid
    rows_per_w = params_v.shape[0]
    n_chunks = rows_per_w // _CH
    row_base = wid * rows_per_w

    pltpu.sync_copy(params_hbm.at[pl.ds(row_base * 1, rows_per_w)], params_v)
    pltpu.sync_copy(g0inv_hbm, g0inv_v)
    pltpu.sync_copy(zeros_hbm, buf0)
    pltpu.sync_copy(zeros_hbm, buf1)

    g0 = g0inv_v[pl.ds(0, _LANES)]
    inv_dx = g0inv_v[pl.ds(_LANES, _LANES)]
    lane512 = jnp.arange(_LANES, dtype=jnp.int32) * _N_GRID
    zeros16 = jnp.zeros((_LANES,), jnp.float32)

    bufs = (buf0, buf1)
    idxs = (idx0, idx1)
    sems = (sem0, sem1)
    copies = [None, None]

    for k in range(n_chunks):
        b = k % 2
        buf, idxb, sem = bufs[b], idxs[b], sems[b]
        if copies[b] is not None:
            copies[b].wait()
            for j in range(_CH // _LANES):
                iv = idxb[pl.ds(j * _LANES, _LANES)]
                plsc.store_scatter(buf, [iv], zeros16)
                plsc.store_scatter(buf, [iv + 1], zeros16)
        for j in range(_CH // _LANES):
            x = params_v[pl.ds(k * _CH + j * _LANES, _LANES)]
            t = (x - g0) * inv_dx
            ind = jnp.minimum(t.astype(jnp.int32) + 1, _N_GRID - 1)
            w0 = ind.astype(jnp.float32) - t
            w1 = 1.0 - w0
            flat0 = lane512 + (ind + (j * _LANES * _N_GRID - 1))
            plsc.store_scatter(buf, [flat0], w0)
            plsc.store_scatter(buf, [flat0 + 1], w1)
            idxb[pl.ds(j * _LANES, _LANES)] = flat0
        copies[b] = pltpu.async_copy(
            buf, out_hbm.at[pl.ds((row_base + k * _CH) * _N_GRID, _TILE)], sem)
    copies[0].wait()
    copies[1].wait()


@jax.jit
def kernel(params, log_tau):
    n_rows = params.shape[0]
    n_grid = log_tau.shape[0]
    g0 = log_tau[0]
    dx = (log_tau[-1] - log_tau[0]) / (n_grid - 1)
    g0inv = jnp.concatenate(
        [jnp.full((_LANES,), g0), jnp.full((_LANES,), 1.0 / dx)])
    zeros_tile = jnp.zeros((_TILE,), jnp.float32)
    rows_per_w = n_rows // _NW

    mesh = plsc.VectorSubcoreMesh(core_axis_name="c", subcore_axis_name="s")
    sc_call = functools.partial(
        pl.kernel,
        mesh=mesh,
        out_type=jax.ShapeDtypeStruct((n_rows * n_grid,), jnp.float32),
        scratch_types=[
            pltpu.VMEM((2 * _LANES,), jnp.float32),
            pltpu.VMEM((rows_per_w,), jnp.float32),
            pltpu.VMEM((_TILE,), jnp.float32),
            pltpu.VMEM((_TILE,), jnp.float32),
            pltpu.VMEM((_CH,), jnp.int32),
            pltpu.VMEM((_CH,), jnp.int32),
            pltpu.SemaphoreType.DMA,
            pltpu.SemaphoreType.DMA,
        ],
        compiler_params=pltpu.CompilerParams(needs_layout_passes=False),
    )(_sc_body)
    flat = sc_call(g0inv, zeros_tile, params.reshape(-1))
    return flat.reshape(n_rows, n_grid)


# speedup vs baseline: 2.6516x; 2.6516x over previous
import functools

import jax
import jax.numpy as jnp
from jax import lax
from jax.experimental import pallas as pl
from jax.experimental.pallas import tpu as pltpu
from jax.experimental.pallas import tpu_sc as plsc

_NW = 32
_LANES = 16
_CH = 64
_N_GRID = 512


def _sc_body(g0inv_hbm, zeros_hbm, params_hbm, out_hbm,
             g0inv_v, params_v, buf0, buf1, idx0, idx1, sem0, sem1):
    cid = lax.axis_index("c")
    sid = lax.axis_index("s")
    wid = sid * 2 + cid
    rows_per_w = params_v.shape[0]
    n_chunks = rows_per_w // _CH
    row_base = wid * rows_per_w

    pltpu.sync_copy(params_hbm.at[pl.ds(row_base * 1, rows_per_w)], params_v)
    pltpu.sync_copy(g0inv_hbm, g0inv_v)
    pltpu.sync_copy(zeros_hbm, buf0)
    pltpu.sync_copy(zeros_hbm, buf1)

    g0 = g0inv_v[pl.ds(0, _LANES)]
    inv_dx = g0inv_v[pl.ds(_LANES, _LANES)]
    lane = jnp.arange(_LANES, dtype=jnp.int32)
    zeros16 = jnp.zeros((_LANES,), jnp.float32)

    bufs = (buf0, buf1)
    idxs = (idx0, idx1)
    sems = (sem0, sem1)
    copies = [None, None]

    for k in range(n_chunks):
        b = k % 2
        buf, idxb, sem = bufs[b], idxs[b], sems[b]
        if copies[b] is not None:
            copies[b].wait()
            for j in range(_CH // _LANES):
                rvec = lane + (j * _LANES)
                cv = idxb[pl.ds(j * _LANES, _LANES)]
                plsc.store_scatter(buf, [rvec, cv], zeros16)
                plsc.store_scatter(buf, [rvec, cv + 1], zeros16)
        for j in range(_CH // _LANES):
            x = params_v[pl.ds(k * _CH + j * _LANES, _LANES)]
            t = (x - g0) * inv_dx
            ind = jnp.minimum(t.astype(jnp.int32) + 1, _N_GRID - 1)
            w0 = ind.astype(jnp.float32) - t
            w1 = 1.0 - w0
            rvec = lane + (j * _LANES)
            cv = ind - 1
            plsc.store_scatter(buf, [rvec, cv], w0)
            plsc.store_scatter(buf, [rvec, cv + 1], w1)
            idxb[pl.ds(j * _LANES, _LANES)] = cv
        copies[b] = pltpu.async_copy(
            buf, out_hbm.at[pl.ds(row_base + k * _CH, _CH)], sem)
    copies[0].wait()
    copies[1].wait()


@jax.jit
def kernel(params, log_tau):
    n_rows = params.shape[0]
    n_grid = log_tau.shape[0]
    g0 = log_tau[0]
    dx = (log_tau[-1] - log_tau[0]) / (n_grid - 1)
    g0inv = jnp.concatenate(
        [jnp.full((_LANES,), g0), jnp.full((_LANES,), 1.0 / dx)])
    zeros_tile = jnp.zeros((_CH, n_grid), jnp.float32)
    rows_per_w = n_rows // _NW

    mesh = plsc.VectorSubcoreMesh(core_axis_name="c", subcore_axis_name="s")
    sc_call = functools.partial(
        pl.kernel,
        mesh=mesh,
        out_type=jax.ShapeDtypeStruct((n_rows, n_grid), jnp.float32),
        scratch_types=[
            pltpu.VMEM((2 * _LANES,), jnp.float32),
            pltpu.VMEM((rows_per_w,), jnp.float32),
            pltpu.VMEM((_CH, n_grid), jnp.float32),
            pltpu.VMEM((_CH, n_grid), jnp.float32),
            pltpu.VMEM((_CH,), jnp.int32),
            pltpu.VMEM((_CH,), jnp.int32),
            pltpu.SemaphoreType.DMA,
            pltpu.SemaphoreType.DMA,
        ],
        compiler_params=pltpu.CompilerParams(needs_layout_passes=False),
    )(_sc_body)
    return sc_call(g0inv, zeros_tile, params.reshape(-1))
